# baseline (speedup 1.0000x reference)
import jax
import jax.numpy as jnp
from jax.experimental import pallas as pl
from jax.experimental.pallas import tpu as pltpu

_C = 128
_G = 2

_MM = (((1,), (0,)), ((), ()))
_MT = (((1,), (1,)), ((), ()))
_TM = (((0,), (0,)), ((), ()))


def _d(a, b, dims):
    return jax.lax.dot_general(a, b, dims,
                               preferred_element_type=jnp.float32)


def _split(x):
    hi = x.astype(jnp.bfloat16)
    lo = (x - hi.astype(jnp.float32)).astype(jnp.bfloat16)
    return hi, lo


def _mm3(ap, bp, dims):
    ahi, alo = ap
    bhi, blo = bp
    return (_d(ahi, bhi, dims) + (_d(ahi, blo, dims) + _d(alo, bhi, dims)))


def _dn_body(x_ref, wqh_ref, wql_ref, wkh_ref, wkl_ref, wvh_ref, wvl_ref,
             woh_ref, wol_ref, bq_ref, bk_ref, bv_ref, bo_ref,
             wbeta_ref, bbeta_ref, o_ref, s_ref):
    G, C, D = x_ref.shape
    ti = pl.program_id(1)

    @pl.when(ti == 0)
    def _():
        s_ref[...] = jnp.zeros_like(s_ref)

    xp = _split(x_ref[...].reshape(G * C, D))
    q_all = _mm3(xp, (wqh_ref[...], wql_ref[...]), _MM) + bq_ref[...]
    k_all = _mm3(xp, (wkh_ref[...], wkl_ref[...]), _MM) + bk_ref[...]
    v_all = _mm3(xp, (wvh_ref[...], wvl_ref[...]), _MM) + bv_ref[...]
    nrm = jnp.sqrt(jnp.sum(k_all * k_all, axis=-1, keepdims=True))
    k_all = k_all / jnp.maximum(nrm, 1e-12)
    beta_all = jax.nn.sigmoid(
        jnp.sum(k_all * wbeta_ref[...], axis=-1, keepdims=True)
        + bbeta_ref[0, 0])

    row = jax.lax.broadcasted_iota(jnp.int32, (C, C), 0)
    col = jax.lax.broadcasted_iota(jnp.int32, (C, C), 1)
    eye = jnp.where(row == col, 1.0, 0.0)
    n_newton = max((C - 1).bit_length() - 2, 0)

    gs = range(G)
    sl = [slice(g * C, (g + 1) * C) for g in gs]
    beta = [beta_all[sl[g]] for g in gs]
    kp = [_split(k_all[sl[g]]) for g in gs]
    a = [jnp.where(row > col, beta[g] * _mm3(kp[g], kp[g], _MT), 0.0)
         for g in gs]
    ap = [_split(a[g]) for g in gs]
    x_inv = [eye - a[g] for g in gs]
    for _ in range(n_newton):
        xb = [x_inv[g].astype(jnp.bfloat16) for g in gs]
        y = [(x_inv[g] + _d(ap[g][0], xb[g], _MM)).astype(jnp.bfloat16)
             for g in gs]
        x_inv = [2.0 * x_inv[g] - _d(xb[g], y[g], _MM) for g in gs]
    xip = [_split(x_inv[g]) for g in gs]
    y = [x_inv[g] + _mm3(ap[g], xip[g], _MM) for g in gs]
    mp = [_split(2.0 * x_inv[g] - _mm3(xip[g], _split(y[g]), _MM))
          for g in gs]

    st = [s_ref[g] for g in gs]
    stp = [_split(st[g]) for g in gs]
    kst = [_mm3(kp[g], stp[g], _MM) for g in gs]
    u = [_mm3(mp[g], _split(beta[g] * (v_all[sl[g]] - kst[g])), _MM)
         for g in gs]
    up = [_split(u[g]) for g in gs]
    qp = [_split(q_all[sl[g]]) for g in gs]
    qk = [jnp.where(row >= col, _mm3(qp[g], kp[g], _MT), 0.0) for g in gs]
    o_acc = [_mm3(qp[g], stp[g], _MM) + _mm3(_split(qk[g]), up[g], _MM)
             for g in gs]
    for g in gs:
        s_ref[g] = st[g] + _mm3(kp[g], up[g], _TM)

    o_all = jnp.concatenate(o_acc, axis=0)
    proj = _mm3(_split(o_all), (woh_ref[...], wol_ref[...]), _MM)
    o_ref[...] = (proj + bo_ref[...]).reshape(G, C, D)


def kernel(x, Wq, bq, Wk, bk, Wv, bv, Wbeta, bbeta, Wo, bo):
    B, T, D = x.shape
    C = _C
    G = _G
    assert T % C == 0 and B % G == 0
    full = lambda b, t: (0, 0)
    wspec = pl.BlockSpec((D, D), full)
    bspec = pl.BlockSpec((1, D), full)
    wqh, wql = _split(Wq.T)
    wkh, wkl = _split(Wk.T)
    wvh, wvl = _split(Wv.T)
    woh, wol = _split(Wo.T)
    return pl.pallas_call(
        _dn_body,
        out_shape=jax.ShapeDtypeStruct((B, T, D), x.dtype),
        grid=(B // G, T // C),
        in_specs=[
            pl.BlockSpec((G, C, D), lambda b, t: (b, t, 0)),
            wspec, wspec, wspec, wspec, wspec, wspec, wspec, wspec,
            bspec, bspec, bspec, bspec, bspec,
            pl.BlockSpec((1, 1), full),
        ],
        out_specs=pl.BlockSpec((G, C, D), lambda b, t: (b, t, 0)),
        scratch_shapes=[pltpu.VMEM((G, D, D), jnp.float32)],
        compiler_params=pltpu.CompilerParams(
            dimension_semantics=("parallel", "arbitrary"),
            vmem_limit_bytes=56 * 1024 * 1024,
        ),
        name="deltanet_chunked",
        interpret=False,
    )(x, wqh, wql, wkh, wkl, wvh, wvl, woh, wol,
      bq.reshape(1, D), bk.reshape(1, D), bv.reshape(1, D),
      bo.reshape(1, D), Wbeta.reshape(1, D), bbeta.reshape(1, 1))


# speedup vs baseline: 1.4086x; 1.4086x over previous
import jax
import jax.numpy as jnp
from jax.experimental import pallas as pl
from jax.experimental.pallas import tpu as pltpu

_C = 128
_G = 4

_MM = (((1,), (0,)), ((), ()))
_MT = (((1,), (1,)), ((), ()))
_TM = (((0,), (0,)), ((), ()))


def _d(a, b, dims):
    return jax.lax.dot_general(a, b, dims,
                               preferred_element_type=jnp.float32)


def _split(x):
    hi = x.astype(jnp.bfloat16)
    lo = (x - hi.astype(jnp.float32)).astype(jnp.bfloat16)
    return hi, lo


def _mm3(ap, bp, dims):
    ahi, alo = ap
    bhi, blo = bp
    return (_d(ahi, bhi, dims) + (_d(ahi, blo, dims) + _d(alo, bhi, dims)))


def _dn_body(x_ref, wqh_ref, wql_ref, wkh_ref, wkl_ref, wvh_ref, wvl_ref,
             woh_ref, wol_ref, bq_ref, bk_ref, bv_ref, bo_ref,
             wbeta_ref, bbeta_ref, o_ref, s_ref):
    G, C, D = x_ref.shape
    ti = pl.program_id(1)

    @pl.when(ti == 0)
    def _():
        s_ref[...] = jnp.zeros_like(s_ref)

    xh = x_ref[...].reshape(G * C, D).astype(jnp.bfloat16)
    q_all = _d(xh, wqh_ref[...], _MM) + bq_ref[...]
    k_all = _d(xh, wkh_ref[...], _MM) + bk_ref[...]
    v_all = _d(xh, wvh_ref[...], _MM) + bv_ref[...]
    nrm = jnp.sqrt(jnp.sum(k_all * k_all, axis=-1, keepdims=True))
    k_all = k_all / jnp.maximum(nrm, 1e-12)
    beta_all = jax.nn.sigmoid(
        jnp.sum(k_all * wbeta_ref[...], axis=-1, keepdims=True)
        + bbeta_ref[0, 0])

    row = jax.lax.broadcasted_iota(jnp.int32, (C, C), 0)
    col = jax.lax.broadcasted_iota(jnp.int32, (C, C), 1)
    eye = jnp.where(row == col, 1.0, 0.0)
    n_newton = max((C - 1).bit_length() - 2, 0)

    gs = range(G)
    sl = [slice(g * C, (g + 1) * C) for g in gs]
    beta = [beta_all[sl[g]] for g in gs]
    kp = [_split(k_all[sl[g]]) for g in gs]
    a = [jnp.where(row > col, beta[g] * _mm3(kp[g], kp[g], _MT), 0.0)
         for g in gs]
    ap = [_split(a[g]) for g in gs]
    x_inv = [eye - a[g] for g in gs]
    for _ in range(n_newton):
        xb = [x_inv[g].astype(jnp.bfloat16) for g in gs]
        y = [(x_inv[g] + _d(ap[g][0], xb[g], _MM)).astype(jnp.bfloat16)
             for g in gs]
        x_inv = [2.0 * x_inv[g] - _d(xb[g], y[g], _MM) for g in gs]
    xip = [_split(x_inv[g]) for g in gs]
    y = [x_inv[g] + _mm3(ap[g], xip[g], _MM) for g in gs]
    mp = [_split(2.0 * x_inv[g] - _mm3(xip[g], _split(y[g]), _MM))
          for g in gs]

    st = [s_ref[g] for g in gs]
    stp = [_split(st[g]) for g in gs]
    kst = [_mm3(kp[g], stp[g], _MM) for g in gs]
    u = [_mm3(mp[g], _split(beta[g] * (v_all[sl[g]] - kst[g])), _MM)
         for g in gs]
    up = [_split(u[g]) for g in gs]
    qp = [_split(q_all[sl[g]]) for g in gs]
    qk = [jnp.where(row >= col, _mm3(qp[g], kp[g], _MT), 0.0) for g in gs]
    o_acc = [_mm3(qp[g], stp[g], _MM) + _mm3(_split(qk[g]), up[g], _MM)
             for g in gs]
    for g in gs:
        s_ref[g] = st[g] + _mm3(kp[g], up[g], _TM)

    o_all = jnp.concatenate(o_acc, axis=0)
    oh = o_all.astype(jnp.bfloat16)
    proj = _d(oh, woh_ref[...], _MM) + _d(oh, wol_ref[...], _MM)
    o_ref[...] = (proj + bo_ref[...]).reshape(G, C, D)


def kernel(x, Wq, bq, Wk, bk, Wv, bv, Wbeta, bbeta, Wo, bo):
    B, T, D = x.shape
    C = _C
    G = _G
    assert T % C == 0 and B % G == 0
    full = lambda b, t: (0, 0)
    wspec = pl.BlockSpec((D, D), full)
    bspec = pl.BlockSpec((1, D), full)
    wqh, wql = _split(Wq.T)
    wkh, wkl = _split(Wk.T)
    wvh, wvl = _split(Wv.T)
    woh, wol = _split(Wo.T)
    return pl.pallas_call(
        _dn_body,
        out_shape=jax.ShapeDtypeStruct((B, T, D), x.dtype),
        grid=(B // G, T // C),
        in_specs=[
            pl.BlockSpec((G, C, D), lambda b, t: (b, t, 0)),
            wspec, wspec, wspec, wspec, wspec, wspec, wspec, wspec,
            bspec, bspec, bspec, bspec, bspec,
            pl.BlockSpec((1, 1), full),
        ],
        out_specs=pl.BlockSpec((G, C, D), lambda b, t: (b, t, 0)),
        scratch_shapes=[pltpu.VMEM((G, D, D), jnp.float32)],
        compiler_params=pltpu.CompilerParams(
            dimension_semantics=("parallel", "arbitrary"),
            vmem_limit_bytes=56 * 1024 * 1024,
        ),
        name="deltanet_chunked",
        interpret=False,
    )(x, wqh, wql, wkh, wkl, wvh, wvl, woh, wol,
      bq.reshape(1, D), bk.reshape(1, D), bv.reshape(1, D),
      bo.reshape(1, D), Wbeta.reshape(1, D), bbeta.reshape(1, 1))


# speedup vs baseline: 1.5329x; 1.0883x over previous
import jax
import jax.numpy as jnp
from jax.experimental import pallas as pl
from jax.experimental.pallas import tpu as pltpu

_C = 128
_G = 4

_MM = (((1,), (0,)), ((), ()))
_MT = (((1,), (1,)), ((), ()))
_TM = (((0,), (0,)), ((), ()))


def _d(a, b, dims):
    return jax.lax.dot_general(a, b, dims,
                               preferred_element_type=jnp.float32)


def _split(x):
    hi = x.astype(jnp.bfloat16)
    lo = (x - hi.astype(jnp.float32)).astype(jnp.bfloat16)
    return hi, lo


def _mm3(ap, bp, dims):
    ahi, alo = ap
    bhi, blo = bp
    return (_d(ahi, bhi, dims) + (_d(ahi, blo, dims) + _d(alo, bhi, dims)))


def _dn_body(x_ref, wqh_ref, wql_ref, wkh_ref, wkl_ref, wvh_ref, wvl_ref,
             woh_ref, wol_ref, bq_ref, bk_ref, bv_ref, bo_ref,
             wbeta_ref, bbeta_ref, o_ref, s_ref):
    G, C, D = x_ref.shape
    ti = pl.program_id(1)

    @pl.when(ti == 0)
    def _():
        s_ref[...] = jnp.zeros_like(s_ref)

    xh = x_ref[...].reshape(G * C, D).astype(jnp.bfloat16)
    q_all = _d(xh, wqh_ref[...], _MM) + bq_ref[...]
    k_all = _d(xh, wkh_ref[...], _MM) + bk_ref[...]
    v_all = _d(xh, wvh_ref[...], _MM) + bv_ref[...]
    nrm = jnp.sqrt(jnp.sum(k_all * k_all, axis=-1, keepdims=True))
    k_all = k_all / jnp.maximum(nrm, 1e-12)
    beta_all = jax.nn.sigmoid(
        jnp.sum(k_all * wbeta_ref[...], axis=-1, keepdims=True)
        + bbeta_ref[0, 0])

    row = jax.lax.broadcasted_iota(jnp.int32, (C, C), 0)
    col = jax.lax.broadcasted_iota(jnp.int32, (C, C), 1)
    eye = jnp.where(row == col, 1.0, 0.0)
    n_newton = max((C - 1).bit_length() - 2, 0)

    gs = range(G)
    sl = [slice(g * C, (g + 1) * C) for g in gs]
    beta = [beta_all[sl[g]] for g in gs]
    kp = [_split(k_all[sl[g]]) for g in gs]
    a = [jnp.where(row > col, beta[g] * _mm3(kp[g], kp[g], _MT), 0.0)
         for g in gs]
    ap = [_split(a[g]) for g in gs]
    x_inv = [eye - a[g] for g in gs]
    for _ in range(n_newton):
        xb = [x_inv[g].astype(jnp.bfloat16) for g in gs]
        y = [(x_inv[g] + _d(ap[g][0], xb[g], _MM)).astype(jnp.bfloat16)
             for g in gs]
        x_inv = [2.0 * x_inv[g] - _d(xb[g], y[g], _MM) for g in gs]
    xip = [_split(x_inv[g]) for g in gs]
    y = [x_inv[g] + _mm3(ap[g], xip[g], _MM) for g in gs]
    mp = [_split(2.0 * x_inv[g] - _mm3(xip[g], _split(y[g]), _MM))
          for g in gs]

    st = [s_ref[g] for g in gs]
    sth = [st[g].astype(jnp.bfloat16) for g in gs]
    kst = [_d(kp[g][0], sth[g], _MM) + _d(kp[g][1], sth[g], _MM)
           for g in gs]
    u = [_mm3(mp[g], _split(beta[g] * (v_all[sl[g]] - kst[g])), _MM)
         for g in gs]
    up = [_split(u[g]) for g in gs]
    qp = [_split(q_all[sl[g]]) for g in gs]
    qk = [jnp.where(row >= col, _mm3(qp[g], kp[g], _MT), 0.0) for g in gs]
    o_acc = [_d(qp[g][0], sth[g], _MM) + _d(qp[g][1], sth[g], _MM)
             + _mm3(_split(qk[g]), up[g], _MM)
             for g in gs]
    for g in gs:
        s_ref[g] = st[g] + _mm3(kp[g], up[g], _TM)

    o_all = jnp.concatenate(o_acc, axis=0)
    oh = o_all.astype(jnp.bfloat16)
    proj = _d(oh, woh_ref[...], _MM) + _d(oh, wol_ref[...], _MM)
    o_ref[...] = (proj + bo_ref[...]).reshape(G, C, D)


def kernel(x, Wq, bq, Wk, bk, Wv, bv, Wbeta, bbeta, Wo, bo):
    B, T, D = x.shape
    C = _C
    G = _G
    assert T % C == 0 and B % G == 0
    full = lambda b, t: (0, 0)
    wspec = pl.BlockSpec((D, D), full)
    bspec = pl.BlockSpec((1, D), full)
    wqh, wql = _split(Wq.T)
    wkh, wkl = _split(Wk.T)
    wvh, wvl = _split(Wv.T)
    woh, wol = _split(Wo.T)
    return pl.pallas_call(
        _dn_body,
        out_shape=jax.ShapeDtypeStruct((B, T, D), x.dtype),
        grid=(B // G, T // C),
        in_specs=[
            pl.BlockSpec((G, C, D), lambda b, t: (b, t, 0)),
            wspec, wspec, wspec, wspec, wspec, wspec, wspec, wspec,
            bspec, bspec, bspec, bspec, bspec,
            pl.BlockSpec((1, 1), full),
        ],
        out_specs=pl.BlockSpec((G, C, D), lambda b, t: (b, t, 0)),
        scratch_shapes=[pltpu.VMEM((G, D, D), jnp.float32)],
        compiler_params=pltpu.CompilerParams(
            dimension_semantics=("parallel", "arbitrary"),
            vmem_limit_bytes=56 * 1024 * 1024,
        ),
        name="deltanet_chunked",
        interpret=False,
    )(x, wqh, wql, wkh, wkl, wvh, wvl, woh, wol,
      bq.reshape(1, D), bk.reshape(1, D), bv.reshape(1, D),
      bo.reshape(1, D), Wbeta.reshape(1, D), bbeta.reshape(1, 1))


# speedup vs baseline: 1.6942x; 1.1052x over previous
import jax
import jax.numpy as jnp
from jax.experimental import pallas as pl
from jax.experimental.pallas import tpu as pltpu

_C = 128
_G = 4

_MM = (((1,), (0,)), ((), ()))
_MT = (((1,), (1,)), ((), ()))
_TM = (((0,), (0,)), ((), ()))


def _d(a, b, dims):
    return jax.lax.dot_general(a, b, dims,
                               preferred_element_type=jnp.float32)


def _split(x):
    hi = x.astype(jnp.bfloat16)
    lo = (x - hi.astype(jnp.float32)).astype(jnp.bfloat16)
    return hi, lo


def _mm3(ap, bp, dims):
    ahi, alo = ap
    bhi, blo = bp
    return (_d(ahi, bhi, dims) + (_d(ahi, blo, dims) + _d(alo, bhi, dims)))


def _dn_body(x_ref, wqh_ref, wql_ref, wkh_ref, wkl_ref, wvh_ref, wvl_ref,
             woh_ref, wol_ref, bq_ref, bk_ref, bv_ref, bo_ref,
             wbeta_ref, bbeta_ref, o_ref, s_ref):
    G, C, D = x_ref.shape
    ti = pl.program_id(1)

    @pl.when(ti == 0)
    def _():
        s_ref[...] = jnp.zeros_like(s_ref)

    xh = x_ref[...].reshape(G * C, D).astype(jnp.bfloat16)
    q_all = _d(xh, wqh_ref[...], _MM) + bq_ref[...]
    k_all = _d(xh, wkh_ref[...], _MM) + bk_ref[...]
    v_all = _d(xh, wvh_ref[...], _MM) + bv_ref[...]
    nrm = jnp.sqrt(jnp.sum(k_all * k_all, axis=-1, keepdims=True))
    k_all = k_all / jnp.maximum(nrm, 1e-12)
    beta_all = jax.nn.sigmoid(
        jnp.sum(k_all * wbeta_ref[...], axis=-1, keepdims=True)
        + bbeta_ref[0, 0])

    row = jax.lax.broadcasted_iota(jnp.int32, (C, C), 0)
    col = jax.lax.broadcasted_iota(jnp.int32, (C, C), 1)
    eye = jnp.where(row == col, 1.0, 0.0)
    n_newton = max((C - 1).bit_length() - 2, 0)

    gs = range(G)
    sl = [slice(g * C, (g + 1) * C) for g in gs]
    beta = [beta_all[sl[g]] for g in gs]
    kp = [_split(k_all[sl[g]]) for g in gs]
    a = [jnp.where(row > col, beta[g] * _mm3(kp[g], kp[g], _MT), 0.0)
         for g in gs]
    ap = [_split(a[g]) for g in gs]
    x_inv = [eye - a[g] for g in gs]
    for _ in range(n_newton):
        xb = [x_inv[g].astype(jnp.bfloat16) for g in gs]
        y = [(x_inv[g] + _d(ap[g][0], xb[g], _MM)).astype(jnp.bfloat16)
             for g in gs]
        x_inv = [2.0 * x_inv[g] - _d(xb[g], y[g], _MM) for g in gs]
    xip = [_split(x_inv[g]) for g in gs]
    y = [x_inv[g] + _mm3(ap[g], xip[g], _MM) for g in gs]
    mp = [_split(2.0 * x_inv[g] - _mm3(xip[g], _split(y[g]), _MM))
          for g in gs]

    st = [s_ref[g] for g in gs]
    sth = [st[g].astype(jnp.bfloat16) for g in gs]
    kst = [_d(kp[g][0], sth[g], _MM) + _d(kp[g][1], sth[g], _MM)
           for g in gs]
    rhs = [_split(beta[g] * (v_all[sl[g]] - kst[g])) for g in gs]
    u = [_d(mp[g][0], rhs[g][0], _MM) + _d(mp[g][0], rhs[g][1], _MM)
         for g in gs]
    up = [_split(u[g]) for g in gs]
    qp = [_split(q_all[sl[g]]) for g in gs]
    qk = [jnp.where(row >= col, _mm3(qp[g], kp[g], _MT), 0.0) for g in gs]
    qkh = [qk[g].astype(jnp.bfloat16) for g in gs]
    o_acc = [_d(qp[g][0], sth[g], _MM) + _d(qp[g][1], sth[g], _MM)
             + _d(qkh[g], up[g][0], _MM) + _d(qkh[g], up[g][1], _MM)
             for g in gs]
    for g in gs:
        s_ref[g] = (st[g] + _d(kp[g][0], up[g][0], _TM)
                    + _d(kp[g][0], up[g][1], _TM))

    o_all = jnp.concatenate(o_acc, axis=0)
    oh = o_all.astype(jnp.bfloat16)
    proj = _d(oh, woh_ref[...], _MM) + _d(oh, wol_ref[...], _MM)
    o_ref[...] = (proj + bo_ref[...]).reshape(G, C, D)


def kernel(x, Wq, bq, Wk, bk, Wv, bv, Wbeta, bbeta, Wo, bo):
    B, T, D = x.shape
    C = _C
    G = _G
    assert T % C == 0 and B % G == 0
    full = lambda b, t: (0, 0)
    wspec = pl.BlockSpec((D, D), full)
    bspec = pl.BlockSpec((1, D), full)
    wqh, wql = _split(Wq.T)
    wkh, wkl = _split(Wk.T)
    wvh, wvl = _split(Wv.T)
    woh, wol = _split(Wo.T)
    return pl.pallas_call(
        _dn_body,
        out_shape=jax.ShapeDtypeStruct((B, T, D), x.dtype),
        grid=(B // G, T // C),
        in_specs=[
            pl.BlockSpec((G, C, D), lambda b, t: (b, t, 0)),
            wspec, wspec, wspec, wspec, wspec, wspec, wspec, wspec,
            bspec, bspec, bspec, bspec, bspec,
            pl.BlockSpec((1, 1), full),
        ],
        out_specs=pl.BlockSpec((G, C, D), lambda b, t: (b, t, 0)),
        scratch_shapes=[pltpu.VMEM((G, D, D), jnp.float32)],
        compiler_params=pltpu.CompilerParams(
            dimension_semantics=("parallel", "arbitrary"),
            vmem_limit_bytes=56 * 1024 * 1024,
        ),
        name="deltanet_chunked",
        interpret=False,
    )(x, wqh, wql, wkh, wkl, wvh, wvl, woh, wol,
      bq.reshape(1, D), bk.reshape(1, D), bv.reshape(1, D),
      bo.reshape(1, D), Wbeta.reshape(1, D), bbeta.reshape(1, 1))


# speedup vs baseline: 1.7206x; 1.0156x over previous
import jax
import jax.numpy as jnp
from jax.experimental import pallas as pl
from jax.experimental.pallas import tpu as pltpu

_C = 128
_G = 4

_MM = (((1,), (0,)), ((), ()))
_MT = (((1,), (1,)), ((), ()))
_TM = (((0,), (0,)), ((), ()))


def _d(a, b, dims):
    return jax.lax.dot_general(a, b, dims,
                               preferred_element_type=jnp.float32)


def _split(x):
    hi = x.astype(jnp.bfloat16)
    lo = (x - hi.astype(jnp.float32)).astype(jnp.bfloat16)
    return hi, lo


def _mm3(ap, bp, dims):
    ahi, alo = ap
    bhi, blo = bp
    return (_d(ahi, bhi, dims) + (_d(ahi, blo, dims) + _d(alo, bhi, dims)))


def _dn_body(x_ref, wqh_ref, wql_ref, wkh_ref, wkl_ref, wvh_ref, wvl_ref,
             woh_ref, wol_ref, bq_ref, bk_ref, bv_ref, bo_ref,
             wbeta_ref, bbeta_ref, o_ref, s_ref):
    G, C, D = x_ref.shape
    ti = pl.program_id(1)

    @pl.when(ti == 0)
    def _():
        s_ref[...] = jnp.zeros_like(s_ref)

    xh = x_ref[...].reshape(G * C, D).astype(jnp.bfloat16)
    q_all = _d(xh, wqh_ref[...], _MM) + bq_ref[...]
    k_all = _d(xh, wkh_ref[...], _MM) + bk_ref[...]
    v_all = _d(xh, wvh_ref[...], _MM) + bv_ref[...]
    nrm = jnp.sqrt(jnp.sum(k_all * k_all, axis=-1, keepdims=True))
    k_all = k_all / jnp.maximum(nrm, 1e-12)
    beta_all = jax.nn.sigmoid(
        jnp.sum(k_all * wbeta_ref[...], axis=-1, keepdims=True)
        + bbeta_ref[0, 0])

    row = jax.lax.broadcasted_iota(jnp.int32, (C, C), 0)
    col = jax.lax.broadcasted_iota(jnp.int32, (C, C), 1)
    eye = jnp.where(row == col, 1.0, 0.0)
    n_newton = max((C - 1).bit_length() - 2, 0)

    gs = range(G)
    sl = [slice(g * C, (g + 1) * C) for g in gs]
    beta = [beta_all[sl[g]] for g in gs]
    kp = [_split(k_all[sl[g]]) for g in gs]
    a = [jnp.where(row > col, beta[g] * _mm3(kp[g], kp[g], _MT), 0.0)
         for g in gs]
    ap = [_split(a[g]) for g in gs]
    x_inv = [eye - a[g] for g in gs]

    st = [s_ref[g] for g in gs]
    sth = [st[g].astype(jnp.bfloat16) for g in gs]
    qp = [_split(q_all[sl[g]]) for g in gs]

    def newton_round(x_inv):
        xb = [x_inv[g].astype(jnp.bfloat16) for g in gs]
        y = [(x_inv[g] + _d(ap[g][0], xb[g], _MM)).astype(jnp.bfloat16)
             for g in gs]
        return [2.0 * x_inv[g] - _d(xb[g], y[g], _MM) for g in gs]

    x_inv = newton_round(x_inv)
    kst = [_d(kp[g][0], sth[g], _MM) + _d(kp[g][1], sth[g], _MM)
           for g in gs]
    x_inv = newton_round(x_inv)
    qst = [_d(qp[g][0], sth[g], _MM) + _d(qp[g][1], sth[g], _MM)
           for g in gs]
    x_inv = newton_round(x_inv)
    qk = [jnp.where(row >= col, _mm3(qp[g], kp[g], _MT), 0.0) for g in gs]
    x_inv = newton_round(x_inv)
    rhs = [_split(beta[g] * (v_all[sl[g]] - kst[g])) for g in gs]
    for _ in range(n_newton - 4):
        x_inv = newton_round(x_inv)
    xip = [_split(x_inv[g]) for g in gs]
    y = [x_inv[g] + _mm3(ap[g], xip[g], _MM) for g in gs]
    mp = [_split(2.0 * x_inv[g] - _mm3(xip[g], _split(y[g]), _MM))
          for g in gs]

    u = [_d(mp[g][0], rhs[g][0], _MM) + _d(mp[g][0], rhs[g][1], _MM)
         for g in gs]
    up = [_split(u[g]) for g in gs]
    qkh = [qk[g].astype(jnp.bfloat16) for g in gs]
    o_acc = [qst[g]
             + _d(qkh[g], up[g][0], _MM) + _d(qkh[g], up[g][1], _MM)
             for g in gs]
    for g in gs:
        s_ref[g] = (st[g] + _d(kp[g][0], up[g][0], _TM)
                    + _d(kp[g][0], up[g][1], _TM))

    o_all = jnp.concatenate(o_acc, axis=0)
    oh = o_all.astype(jnp.bfloat16)
    proj = _d(oh, woh_ref[...], _MM) + _d(oh, wol_ref[...], _MM)
    o_ref[...] = (proj + bo_ref[...]).reshape(G, C, D)


def kernel(x, Wq, bq, Wk, bk, Wv, bv, Wbeta, bbeta, Wo, bo):
    B, T, D = x.shape
    C = _C
    G = _G
    assert T % C == 0 and B % G == 0
    full = lambda b, t: (0, 0)
    wspec = pl.BlockSpec((D, D), full)
    bspec = pl.BlockSpec((1, D), full)
    wqh, wql = _split(Wq.T)
    wkh, wkl = _split(Wk.T)
    wvh, wvl = _split(Wv.T)
    woh, wol = _split(Wo.T)
    return pl.pallas_call(
        _dn_body,
        out_shape=jax.ShapeDtypeStruct((B, T, D), x.dtype),
        grid=(B // G, T // C),
        in_specs=[
            pl.BlockSpec((G, C, D), lambda b, t: (b, t, 0)),
            wspec, wspec, wspec, wspec, wspec, wspec, wspec, wspec,
            bspec, bspec, bspec, bspec, bspec,
            pl.BlockSpec((1, 1), full),
        ],
        out_specs=pl.BlockSpec((G, C, D), lambda b, t: (b, t, 0)),
        scratch_shapes=[pltpu.VMEM((G, D, D), jnp.float32)],
        compiler_params=pltpu.CompilerParams(
            dimension_semantics=("parallel", "arbitrary"),
            vmem_limit_bytes=56 * 1024 * 1024,
        ),
        name="deltanet_chunked",
        interpret=False,
    )(x, wqh, wql, wkh, wkl, wvh, wvl, woh, wol,
      bq.reshape(1, D), bk.reshape(1, D), bv.reshape(1, D),
      bo.reshape(1, D), Wbeta.reshape(1, D), bbeta.reshape(1, 1))


# speedup vs baseline: 2.3640x; 1.3739x over previous
import jax
import jax.numpy as jnp
from jax.experimental import pallas as pl
from jax.experimental.pallas import tpu as pltpu

_C = 128
_G = 4

_MM = (((1,), (0,)), ((), ()))
_MT = (((1,), (1,)), ((), ()))
_TM = (((0,), (0,)), ((), ()))


def _d(a, b, dims):
    return jax.lax.dot_general(a, b, dims,
                               preferred_element_type=jnp.float32)


def _split(x):
    hi = x.astype(jnp.bfloat16)
    lo = (x - hi.astype(jnp.float32)).astype(jnp.bfloat16)
    return hi, lo


def _mm3(ap, bp, dims):
    ahi, alo = ap
    bhi, blo = bp
    return (_d(ahi, bhi, dims) + (_d(ahi, blo, dims) + _d(alo, bhi, dims)))


def _dn_body(x_ref, wqh_ref, wql_ref, wkh_ref, wkl_ref, wvh_ref, wvl_ref,
             woh_ref, wol_ref, bq_ref, bk_ref, bv_ref, bo_ref,
             wbeta_ref, bbeta_ref, o_ref, s_ref):
    G, C, D = x_ref.shape
    ti = pl.program_id(1)

    @pl.when(ti == 0)
    def _():
        s_ref[...] = jnp.zeros_like(s_ref)

    xh = x_ref[...].reshape(G * C, D).astype(jnp.bfloat16)
    q_all = _d(xh, wqh_ref[...], _MM) + bq_ref[...]
    k_all = _d(xh, wkh_ref[...], _MM) + bk_ref[...]
    v_all = _d(xh, wvh_ref[...], _MM) + bv_ref[...]
    nrm = jnp.sqrt(jnp.sum(k_all * k_all, axis=-1, keepdims=True))
    k_all = k_all / jnp.maximum(nrm, 1e-12)
    beta_all = jax.nn.sigmoid(
        jnp.sum(k_all * wbeta_ref[...], axis=-1, keepdims=True)
        + bbeta_ref[0, 0])

    row = jax.lax.broadcasted_iota(jnp.int32, (C, C), 0)
    col = jax.lax.broadcasted_iota(jnp.int32, (C, C), 1)
    eye = jnp.where(row == col, 1.0, 0.0)
    n_newton = max((C - 1).bit_length() - 2, 0)

    gs = range(G)
    sl = [slice(g * C, (g + 1) * C) for g in gs]
    beta = [beta_all[sl[g]] for g in gs]
    kh = [k_all[sl[g]].astype(jnp.bfloat16) for g in gs]
    a = [jnp.where(row > col, beta[g] * _d(kh[g], kh[g], _MT), 0.0)
         for g in gs]
    ah = [a[g].astype(jnp.bfloat16) for g in gs]
    x_inv = [eye - a[g] for g in gs]

    st = [s_ref[g] for g in gs]
    sth = [st[g].astype(jnp.bfloat16) for g in gs]
    qh = [q_all[sl[g]].astype(jnp.bfloat16) for g in gs]

    def newton_round(x_inv):
        xb = [x_inv[g].astype(jnp.bfloat16) for g in gs]
        y = [(x_inv[g] + _d(ah[g], xb[g], _MM)).astype(jnp.bfloat16)
             for g in gs]
        return [2.0 * x_inv[g] - _d(xb[g], y[g], _MM) for g in gs]

    x_inv = newton_round(x_inv)
    kst = [_d(kh[g], sth[g], _MM) for g in gs]
    x_inv = newton_round(x_inv)
    qst = [_d(qh[g], sth[g], _MM) for g in gs]
    x_inv = newton_round(x_inv)
    qk = [jnp.where(row >= col, _d(qh[g], kh[g], _MT), 0.0) for g in gs]
    x_inv = newton_round(x_inv)
    rhsh = [(beta[g] * (v_all[sl[g]] - kst[g])).astype(jnp.bfloat16)
            for g in gs]
    x_inv = newton_round(x_inv)
    x_inv = newton_round(x_inv)
    mh = [x_inv[g].astype(jnp.bfloat16) for g in gs]

    u = [_d(mh[g], rhsh[g], _MM) for g in gs]
    uh = [u[g].astype(jnp.bfloat16) for g in gs]
    qkh = [qk[g].astype(jnp.bfloat16) for g in gs]
    o_acc = [qst[g] + _d(qkh[g], uh[g], _MM) for g in gs]
    for g in gs:
        s_ref[g] = st[g] + _d(kh[g], uh[g], _TM)

    o_all = jnp.concatenate(o_acc, axis=0)
    oh = o_all.astype(jnp.bfloat16)
    proj = _d(oh, woh_ref[...], _MM) + _d(oh, wol_ref[...], _MM)
    o_ref[...] = (proj + bo_ref[...]).reshape(G, C, D)


def kernel(x, Wq, bq, Wk, bk, Wv, bv, Wbeta, bbeta, Wo, bo):
    B, T, D = x.shape
    C = _C
    G = _G
    assert T % C == 0 and B % G == 0
    full = lambda b, t: (0, 0)
    wspec = pl.BlockSpec((D, D), full)
    bspec = pl.BlockSpec((1, D), full)
    wqh, wql = _split(Wq.T)
    wkh, wkl = _split(Wk.T)
    wvh, wvl = _split(Wv.T)
    woh, wol = _split(Wo.T)
    return pl.pallas_call(
        _dn_body,
        out_shape=jax.ShapeDtypeStruct((B, T, D), x.dtype),
        grid=(B // G, T // C),
        in_specs=[
            pl.BlockSpec((G, C, D), lambda b, t: (b, t, 0)),
            wspec, wspec, wspec, wspec, wspec, wspec, wspec, wspec,
            bspec, bspec, bspec, bspec, bspec,
            pl.BlockSpec((1, 1), full),
        ],
        out_specs=pl.BlockSpec((G, C, D), lambda b, t: (b, t, 0)),
        scratch_shapes=[pltpu.VMEM((G, D, D), jnp.float32)],
        compiler_params=pltpu.CompilerParams(
            dimension_semantics=("parallel", "arbitrary"),
            vmem_limit_bytes=56 * 1024 * 1024,
        ),
        name="deltanet_chunked",
        interpret=False,
    )(x, wqh, wql, wkh, wkl, wvh, wvl, woh, wol,
      bq.reshape(1, D), bk.reshape(1, D), bv.reshape(1, D),
      bo.reshape(1, D), Wbeta.reshape(1, D), bbeta.reshape(1, 1))
